# baseline (speedup 1.0000x reference)
import functools

import jax
import jax.numpy as jnp
from jax import lax
from jax.experimental import pallas as pl
from jax.experimental.pallas import tpu as pltpu
from jax.experimental.pallas import tpu_sc as plsc

NUM_EMB = 100000
D = 128
B_TOK, S = 4096, 200
B = B_TOK * S
B_SC = 737280
NC, NS = 2, 16
NW = NC * NS
BPW = B_SC // NW
C = 128
NCHUNK = BPW // C
GPB = 2
NGRP = NCHUNK // GPB
NBUF = 3
K = 2

_HEAD = NBUF
_STEADY = (NGRP - 2 * NBUF) // NBUF
_TAIL_START = _HEAD + NBUF * _STEADY

_mesh = plsc.VectorSubcoreMesh(core_axis_name="c", subcore_axis_name="s")


@functools.partial(
    pl.kernel,
    mesh=_mesh,
    out_type=jax.ShapeDtypeStruct((B_SC, D), jnp.float32),
    scratch_types=[
        pltpu.VMEM((NCHUNK, C), jnp.int32),
        pltpu.VMEM((NBUF, GPB * C, D), jnp.float32),
    ] + [pltpu.SemaphoreType.DMA] * (2 * NBUF),
)
def _emb_lookup(idx_hbm, tab_hbm, out_hbm, idx_v, rows_v, *sems):
    wid = lax.axis_index("s") * NC + lax.axis_index("c")
    base = wid * BPW
    gsem = sems[:NBUF]
    wsem = sems[NBUF:]
    pltpu.sync_copy(idx_hbm.at[wid], idx_v)

    def issue_g(g, b):
        for s in range(GPB):
            pltpu.async_copy(
                tab_hbm.at[idx_v.at[g * GPB + s]],
                rows_v.at[b, pl.ds(s * C, C)],
                gsem[b],
            )

    def drain_g(g, b):
        for s in range(GPB):
            pltpu.make_async_copy(
                tab_hbm.at[idx_v.at[g * GPB + s]],
                rows_v.at[b, pl.ds(s * C, C)],
                gsem[b],
            ).wait()

    def issue_w(g, b):
        pltpu.async_copy(
            rows_v.at[b], out_hbm.at[pl.ds(base + g * GPB * C, GPB * C)], wsem[b]
        )

    def wait_w(g, b):
        pltpu.make_async_copy(
            rows_v.at[b], out_hbm.at[pl.ds(base + g * GPB * C, GPB * C)], wsem[b]
        ).wait()

    for g in range(K):
        issue_g(g, g % NBUF)

    def group_step(g, b):
        b2 = (b + K) % NBUF
        drain_g(g, b)
        issue_w(g, b)
        wait_w(g + K - NBUF, b2)
        issue_g(g + K, b2)

    for g in range(_HEAD):
        b = g % NBUF
        drain_g(g, b)
        issue_w(g, b)
        if g + K - NBUF >= 0:
            wait_w(g + K - NBUF, (g + K) % NBUF)
        issue_g(g + K, (g + K) % NBUF)

    def body(jj, carry):
        for bb in range(NBUF):
            g = _HEAD + NBUF * jj + bb
            group_step(g, (_HEAD + bb) % NBUF)
        return carry

    lax.fori_loop(0, _STEADY, body, 0)

    for g in range(_TAIL_START, NGRP):
        b = g % NBUF
        drain_g(g, b)
        issue_w(g, b)
        if g + K < NGRP:
            wait_w(g + K - NBUF, (g + K) % NBUF)
            issue_g(g + K, (g + K) % NBUF)

    for g in range(NGRP - NBUF, NGRP):
        wait_w(g, g % NBUF)


def kernel(token_ids, embeddings):
    flat = token_ids.reshape(B)
    idx_sc = flat[:B_SC].reshape(NW, NCHUNK, C)
    out_sc = _emb_lookup(idx_sc, embeddings)
    out_tc = jnp.take(embeddings, flat[B_SC:], axis=0)
    return jnp.concatenate([out_sc, out_tc], axis=0).reshape(B_TOK, S, D)


# speedup vs baseline: 1.8953x; 1.8953x over previous
import functools

import jax
import jax.numpy as jnp
from jax import lax
from jax.experimental import pallas as pl
from jax.experimental.pallas import tpu as pltpu
from jax.experimental.pallas import tpu_sc as plsc

NUM_EMB = 100000
D = 128
B_TOK, S = 4096, 200
B = B_TOK * S
B_SC = B
NC, NS = 2, 16
NW = NC * NS
BPW = B_SC // NW
C = 128
NCHUNK = BPW // C
GPB = 2
NGRP = NCHUNK // GPB
NBUF = 3
K = 2

_HEAD = NBUF
_STEADY = (NGRP - 2 * NBUF) // NBUF
_TAIL_START = _HEAD + NBUF * _STEADY

_mesh = plsc.VectorSubcoreMesh(core_axis_name="c", subcore_axis_name="s")


@functools.partial(
    pl.kernel,
    mesh=_mesh,
    out_type=jax.ShapeDtypeStruct((B_SC, D), jnp.float32),
    scratch_types=[
        pltpu.VMEM((NCHUNK, C), jnp.int32),
        pltpu.VMEM((NBUF, GPB * C, D), jnp.float32),
    ] + [pltpu.SemaphoreType.DMA] * (2 * NBUF),
)
def _emb_lookup(idx_hbm, tab_hbm, out_hbm, idx_v, rows_v, *sems):
    wid = lax.axis_index("s") * NC + lax.axis_index("c")
    base = wid * BPW
    gsem = sems[:NBUF]
    wsem = sems[NBUF:]
    pltpu.sync_copy(idx_hbm.at[wid], idx_v)

    def issue_g(g, b):
        for s in range(GPB):
            pltpu.async_copy(
                tab_hbm.at[idx_v.at[g * GPB + s]],
                rows_v.at[b, pl.ds(s * C, C)],
                gsem[b],
            )

    def drain_g(g, b):
        for s in range(GPB):
            pltpu.make_async_copy(
                tab_hbm.at[idx_v.at[g * GPB + s]],
                rows_v.at[b, pl.ds(s * C, C)],
                gsem[b],
            ).wait()

    def issue_w(g, b):
        pltpu.async_copy(
            rows_v.at[b], out_hbm.at[pl.ds(base + g * GPB * C, GPB * C)], wsem[b]
        )

    def wait_w(g, b):
        pltpu.make_async_copy(
            rows_v.at[b], out_hbm.at[pl.ds(base + g * GPB * C, GPB * C)], wsem[b]
        ).wait()

    for g in range(K):
        issue_g(g, g % NBUF)

    def group_step(g, b):
        b2 = (b + K) % NBUF
        drain_g(g, b)
        issue_w(g, b)
        wait_w(g + K - NBUF, b2)
        issue_g(g + K, b2)

    for g in range(_HEAD):
        b = g % NBUF
        drain_g(g, b)
        issue_w(g, b)
        if g + K - NBUF >= 0:
            wait_w(g + K - NBUF, (g + K) % NBUF)
        issue_g(g + K, (g + K) % NBUF)

    def body(jj, carry):
        for bb in range(NBUF):
            g = _HEAD + NBUF * jj + bb
            group_step(g, (_HEAD + bb) % NBUF)
        return carry

    lax.fori_loop(0, _STEADY, body, 0)

    for g in range(_TAIL_START, NGRP):
        b = g % NBUF
        drain_g(g, b)
        issue_w(g, b)
        if g + K < NGRP:
            wait_w(g + K - NBUF, (g + K) % NBUF)
            issue_g(g + K, (g + K) % NBUF)

    for g in range(NGRP - NBUF, NGRP):
        wait_w(g, g % NBUF)


def kernel(token_ids, embeddings):
    idx = token_ids.reshape(NW, NCHUNK, C)
    out = _emb_lookup(idx, embeddings)
    return out.reshape(B_TOK, S, D)


# speedup vs baseline: 1.9098x; 1.0076x over previous
import functools

import jax
import jax.numpy as jnp
from jax import lax
from jax.experimental import pallas as pl
from jax.experimental.pallas import tpu as pltpu
from jax.experimental.pallas import tpu_sc as plsc

NUM_EMB = 100000
D = 128
B_TOK, S = 4096, 200
B = B_TOK * S
B_SC = B
NC, NS = 2, 16
NW = NC * NS
BPW = B_SC // NW
C = 128
NCHUNK = BPW // C
GPB = 1
NGRP = NCHUNK // GPB
NBUF = 2
K = 2

_HEAD = NBUF
_STEADY = (NGRP - 2 * NBUF) // NBUF
_TAIL_START = _HEAD + NBUF * _STEADY

_mesh = plsc.VectorSubcoreMesh(core_axis_name="c", subcore_axis_name="s")


@functools.partial(
    pl.kernel,
    mesh=_mesh,
    out_type=jax.ShapeDtypeStruct((B_SC, D), jnp.float32),
    scratch_types=[
        pltpu.VMEM((NCHUNK, C), jnp.int32),
        pltpu.VMEM((NBUF, GPB * C, D), jnp.float32),
    ] + [pltpu.SemaphoreType.DMA] * (2 * NBUF),
)
def _emb_lookup(idx_hbm, tab_hbm, out_hbm, idx_v, rows_v, *sems):
    wid = lax.axis_index("s") * NC + lax.axis_index("c")
    base = wid * BPW
    gsem = sems[:NBUF]
    wsem = sems[NBUF:]
    pltpu.sync_copy(idx_hbm.at[wid], idx_v)

    def issue_g(g, b):
        for s in range(GPB):
            pltpu.async_copy(
                tab_hbm.at[idx_v.at[g * GPB + s]],
                rows_v.at[b, pl.ds(s * C, C)],
                gsem[b],
            )

    def drain_g(g, b):
        for s in range(GPB):
            pltpu.make_async_copy(
                tab_hbm.at[idx_v.at[g * GPB + s]],
                rows_v.at[b, pl.ds(s * C, C)],
                gsem[b],
            ).wait()

    def issue_w(g, b):
        pltpu.async_copy(
            rows_v.at[b], out_hbm.at[pl.ds(base + g * GPB * C, GPB * C)], wsem[b]
        )

    def wait_w(g, b):
        pltpu.make_async_copy(
            rows_v.at[b], out_hbm.at[pl.ds(base + g * GPB * C, GPB * C)], wsem[b]
        ).wait()

    for g in range(K):
        issue_g(g, g % NBUF)

    def group_step(g, b):
        b2 = (b + K) % NBUF
        drain_g(g, b)
        issue_w(g, b)
        wait_w(g + K - NBUF, b2)
        issue_g(g + K, b2)

    for g in range(_HEAD):
        b = g % NBUF
        drain_g(g, b)
        issue_w(g, b)
        if g + K - NBUF >= 0:
            wait_w(g + K - NBUF, (g + K) % NBUF)
        issue_g(g + K, (g + K) % NBUF)

    def body(jj, carry):
        for bb in range(NBUF):
            g = _HEAD + NBUF * jj + bb
            group_step(g, (_HEAD + bb) % NBUF)
        return carry

    lax.fori_loop(0, _STEADY, body, 0)

    for g in range(_TAIL_START, NGRP):
        b = g % NBUF
        drain_g(g, b)
        issue_w(g, b)
        if g + K < NGRP:
            wait_w(g + K - NBUF, (g + K) % NBUF)
            issue_g(g + K, (g + K) % NBUF)

    for g in range(NGRP - NBUF, NGRP):
        wait_w(g, g % NBUF)


def kernel(token_ids, embeddings):
    idx = token_ids.reshape(NW, NCHUNK, C)
    out = _emb_lookup(idx, embeddings)
    return out.reshape(B_TOK, S, D)
